# baseline (speedup 1.0000x reference)
import jax
import jax.numpy as jnp
from jax import lax
from jax.experimental import pallas as pl
from jax.experimental.pallas import tpu as pltpu
from jax.experimental.pallas import tpu_sc as plsc

N = 10000
C = 128
H = 4
FF = 512
E = 160000
EE = E + N
NTILES = 16
NCORES = 2
CHUNK = 64
CHUNKS_PER_TILE = 168
EP = NTILES * CHUNKS_PER_TILE * CHUNK
ACC_N = N + 112
ROWS_PER_TILE = ACC_N // NTILES
DW = 16
BN = 1000
GRID = N // BN


def _pre_body(x_ref, wg_ref, asrc_ref, adst_ref, h_ref, ad_ref, g_ref, macc_ref):
    i = pl.program_id(0)
    xb = x_ref[...]
    hb = jnp.dot(xb, wg_ref[...], preferred_element_type=jnp.float32)
    h_ref[...] = hb
    for hh in range(H):
        hcol = hb[:, hh * C:(hh + 1) * C]
        av = jnp.sum(hcol * asrc_ref[hh:hh + 1, :], axis=1)
        dv = jnp.sum(hcol * adst_ref[hh:hh + 1, :], axis=1)
        ad_ref[0, hh:hh + 1, :] = av.reshape(1, BN)
        ad_ref[0, H + hh:H + hh + 1, :] = dv.reshape(1, BN)
        ms = jnp.max(av)
        md = jnp.max(dv)

        @pl.when(i == 0)
        def _():
            macc_ref[hh:hh + 1, :] = jnp.full((1, 128), ms, jnp.float32)
            macc_ref[H + hh:H + hh + 1, :] = jnp.full((1, 128), md, jnp.float32)

        @pl.when(i > 0)
        def _():
            macc_ref[hh:hh + 1, :] = jnp.maximum(macc_ref[hh:hh + 1, :], ms)
            macc_ref[H + hh:H + hh + 1, :] = jnp.maximum(
                macc_ref[H + hh:H + hh + 1, :], md)

    @pl.when(i == GRID - 1)
    def _():
        g_ref[...] = jnp.concatenate(
            [jnp.maximum(macc_ref[0:H, :] + macc_ref[H:2 * H, :], 0.0),
             jnp.zeros((8 - H, 128), jnp.float32)], axis=0)


def _pre(x, Wg, att_src, att_dst):
    return pl.pallas_call(
        _pre_body,
        grid=(GRID,),
        in_specs=[
            pl.BlockSpec((BN, C), lambda i: (i, 0)),
            pl.BlockSpec((C, H * C), lambda i: (0, 0)),
            pl.BlockSpec((H, C), lambda i: (0, 0)),
            pl.BlockSpec((H, C), lambda i: (0, 0)),
        ],
        out_specs=[
            pl.BlockSpec((BN, H * C), lambda i: (i, 0)),
            pl.BlockSpec((1, 2 * H, BN), lambda i: (i, 0, 0)),
            pl.BlockSpec((2 * H, 128), lambda i: (0, 0)),
        ],
        out_shape=[
            jax.ShapeDtypeStruct((N, H * C), jnp.float32),
            jax.ShapeDtypeStruct((GRID, 2 * H, BN), jnp.float32),
            jax.ShapeDtypeStruct((2 * H, 128), jnp.float32),
        ],
        scratch_shapes=[pltpu.VMEM((2 * H, 128), jnp.float32)],
    )(x, Wg, att_src, att_dst)


def _edge_body(h40, adbuf, gbuf, srcI, dstI, accout, denout,
               as_l, ad_l, g_l, src_v, dst_v, idx_v, sidx_v, ea_v,
               rows_v, den_v, acc_sh, den_sh,
               gsem, isem_s, isem_d, ssem_a, ssem_d):
    c = lax.axis_index("c")
    s = lax.axis_index("s")
    onehot = jnp.where(lax.iota(jnp.int32, 16) == 0, 1.0, 0.0)
    lanes = [jnp.full((16,), i, jnp.int32) for i in range(16)]
    base = s * ROWS_PER_TILE
    CPT = CHUNKS_PER_TILE

    def _idx_compute(nb, hd):
        for k in range(CHUNK // 16):
            s16 = src_v[nb, pl.ds(16 * k, 16)]
            idx_v[nb, pl.ds(16 * k, 16)] = s16 * H + hd

    if True:
        for r in range(2):
            hd = r * NCORES + c
            for g in range(GRID):
                pltpu.sync_copy(adbuf.at[g].at[hd], as_l.at[pl.ds(g * BN, BN)])
                pltpu.sync_copy(adbuf.at[g].at[hd + H],
                                ad_l.at[pl.ds(g * BN, BN)])
            as_l[pl.ds(N, 16)] = jnp.zeros((16,), jnp.float32)
            ad_l[pl.ds(N, 16)] = jnp.zeros((16,), jnp.float32)
            pltpu.sync_copy(gbuf.at[hd], g_l)
            gvec = g_l[pl.ds(0, 16)]

            def zrow(rr, carry):
                for k in range(8):
                    rows_v[0, rr, pl.ds(16 * k, 16)] = jnp.zeros(
                        (16,), jnp.float32)
                den_v[0, rr, :] = jnp.zeros((16,), jnp.float32)
                return carry

            lax.fori_loop(0, CHUNK, zrow, 0)
            for k in range(ROWS_PER_TILE // CHUNK):
                pltpu.sync_copy(rows_v.at[0],
                                acc_sh.at[pl.ds(base + k * CHUNK, CHUNK)])
                pltpu.sync_copy(den_v.at[0],
                                den_sh.at[pl.ds(base + k * CHUNK, CHUNK)])
            rem = ROWS_PER_TILE % CHUNK
            pltpu.sync_copy(
                rows_v.at[0].at[pl.ds(0, rem)],
                acc_sh.at[pl.ds(base + ROWS_PER_TILE - rem, rem)])
            pltpu.sync_copy(
                den_v.at[0].at[pl.ds(0, rem)],
                den_sh.at[pl.ds(base + ROWS_PER_TILE - rem, rem)])
            plsc.subcore_barrier()

            pltpu.sync_copy(srcI.at[s * CPT], src_v.at[0])
            pltpu.sync_copy(dstI.at[s * CPT], dst_v.at[0])
            _idx_compute(0, hd)
            pltpu.async_copy(h40.at[idx_v.at[0]], rows_v.at[0], gsem.at[0])
            pltpu.async_copy(srcI.at[s * CPT + 1], src_v.at[1], isem_s.at[1])
            pltpu.async_copy(dstI.at[s * CPT + 1], dst_v.at[1], isem_d.at[1])

            def chunk_body(j, carry):
                bb = lax.rem(j, 2)
                nb = 1 - bb
                pltpu.make_async_copy(
                    h40.at[idx_v.at[bb]], rows_v.at[bb], gsem.at[bb]).wait()

                @pl.when(j < CPT - 1)
                def _():
                    pltpu.make_async_copy(
                        srcI.at[0], src_v.at[nb], isem_s.at[nb]).wait()
                    pltpu.make_async_copy(
                        dstI.at[0], dst_v.at[nb], isem_d.at[nb]).wait()
                    _idx_compute(nb, hd)

                    @pl.when(j > 0)
                    def _():
                        pltpu.make_async_copy(
                            rows_v.at[nb], acc_sh.at[sidx_v.at[nb]],
                            ssem_a.at[nb]).wait()
                        pltpu.make_async_copy(
                            den_v.at[nb], den_sh.at[sidx_v.at[nb]],
                            ssem_d.at[nb]).wait()

                    pltpu.async_copy(h40.at[idx_v.at[nb]], rows_v.at[nb],
                                     gsem.at[nb])

                for g2 in range(CHUNK // 16):
                    s16 = src_v[bb, pl.ds(16 * g2, 16)]
                    d16 = dst_v[bb, pl.ds(16 * g2, 16)]
                    al = (plsc.load_gather(as_l, [s16])
                          + plsc.load_gather(ad_l, [d16]))
                    al = jnp.maximum(al, al * 0.2)
                    eavec = jnp.exp(al - gvec)
                    ea_v[pl.ds(16 * g2, 16)] = eavec
                    sidx_v[bb, pl.ds(16 * g2, 16)] = d16
                    for i in range(16):
                        rr = g2 * 16 + i
                        eav = plsc.load_gather(ea_v, [lanes[0] + rr])
                        for k in range(8):
                            rows_v[bb, rr, pl.ds(16 * k, 16)] = (
                                rows_v[bb, rr, pl.ds(16 * k, 16)] * eav)
                        den_v[bb, rr, :] = eav * onehot
                pltpu.async_copy(rows_v.at[bb], acc_sh.at[sidx_v.at[bb]],
                                 ssem_a.at[bb], add=True)
                pltpu.async_copy(den_v.at[bb], den_sh.at[sidx_v.at[bb]],
                                 ssem_d.at[bb], add=True)
                @pl.when(j < CPT - 2)
                def _():
                    row2 = s * CPT + j + 2
                    pltpu.async_copy(srcI.at[row2], src_v.at[bb],
                                     isem_s.at[bb])
                    pltpu.async_copy(dstI.at[row2], dst_v.at[bb],
                                     isem_d.at[bb])
                return carry

            lax.fori_loop(0, CPT, chunk_body, 0)
            for p in ((CPT - 2) % 2, (CPT - 1) % 2):
                pltpu.make_async_copy(
                    rows_v.at[p], acc_sh.at[sidx_v.at[p]], ssem_a.at[p]).wait()
                pltpu.make_async_copy(
                    den_v.at[p], den_sh.at[sidx_v.at[p]], ssem_d.at[p]).wait()
            plsc.subcore_barrier()
            pltpu.sync_copy(acc_sh.at[pl.ds(base, ROWS_PER_TILE)],
                            accout.at[hd].at[pl.ds(base, ROWS_PER_TILE)])
            pltpu.sync_copy(den_sh.at[pl.ds(base, ROWS_PER_TILE)],
                            denout.at[hd].at[pl.ds(base, ROWS_PER_TILE)])
            plsc.subcore_barrier()


def _edge(h40, adbuf, gbuf, src_p, dst_p):
    mesh = plsc.VectorSubcoreMesh(core_axis_name="c", subcore_axis_name="s",
                                  num_cores=NCORES, num_subcores=NTILES)
    fn = pl.kernel(
        _edge_body,
        out_type=[
            jax.ShapeDtypeStruct((H, ACC_N, C), jnp.float32),
            jax.ShapeDtypeStruct((H, ACC_N, DW), jnp.float32),
        ],
        mesh=mesh,
        compiler_params=pltpu.CompilerParams(
            needs_layout_passes=False, use_tc_tiling_on_sc=False),
        scratch_types=[
            pltpu.VMEM((N + 16,), jnp.float32),
            pltpu.VMEM((N + 16,), jnp.float32),
            pltpu.VMEM((128,), jnp.float32),
            pltpu.VMEM((2, CHUNK), jnp.int32),
            pltpu.VMEM((2, CHUNK), jnp.int32),
            pltpu.VMEM((2, CHUNK), jnp.int32),
            pltpu.VMEM((2, CHUNK), jnp.int32),
            pltpu.VMEM((CHUNK,), jnp.float32),
            pltpu.VMEM((2, CHUNK, C), jnp.float32),
            pltpu.VMEM((2, CHUNK, DW), jnp.float32),
            pltpu.VMEM_SHARED((ACC_N, C), jnp.float32),
            pltpu.VMEM_SHARED((ACC_N, DW), jnp.float32),
            pltpu.SemaphoreType.DMA((2,)),
            pltpu.SemaphoreType.DMA((2,)),
            pltpu.SemaphoreType.DMA((2,)),
            pltpu.SemaphoreType.DMA((2,)),
            pltpu.SemaphoreType.DMA((2,)),
        ],
    )
    return fn(h40, adbuf, gbuf, src_p, dst_p)


def _post_body(acc_ref, den_ref, x_ref, bg_ref, w1_ref, b1_ref, w2_ref,
               b2_ref, g1_ref, be1_ref, g2_ref, be2_ref, o_ref):
    num = acc_ref[...]
    den = den_ref[:, :, 0:1]
    outv = jnp.sum(num / (den + 1e-16), axis=0) * (1.0 / H) + bg_ref[...]
    t = x_ref[...] + outv
    mu = jnp.mean(t, axis=-1, keepdims=True)
    var = jnp.mean((t - mu) ** 2, axis=-1, keepdims=True)
    x1 = (t - mu) / jnp.sqrt(var + 1e-5) * g1_ref[...] + be1_ref[...]
    f = jnp.maximum(
        jnp.dot(x1, w1_ref[...], preferred_element_type=jnp.float32)
        + b1_ref[...], 0.0)
    f2 = (jnp.dot(f, w2_ref[...], preferred_element_type=jnp.float32)
          + b2_ref[...])
    t2 = x1 + f2
    mu2 = jnp.mean(t2, axis=-1, keepdims=True)
    var2 = jnp.mean((t2 - mu2) ** 2, axis=-1, keepdims=True)
    o_ref[...] = (t2 - mu2) / jnp.sqrt(var2 + 1e-5) * g2_ref[...] + be2_ref[...]


def _post(accp, denp, x, bg, W1, b1, W2, b2, g1, be1, g2, be2):
    def vec():
        return pl.BlockSpec((1, C), lambda i: (0, 0))
    return pl.pallas_call(
        _post_body,
        grid=(GRID,),
        in_specs=[
            pl.BlockSpec((H, BN, C), lambda i: (0, i, 0)),
            pl.BlockSpec((H, BN, DW), lambda i: (0, i, 0)),
            pl.BlockSpec((BN, C), lambda i: (i, 0)),
            vec(),
            pl.BlockSpec((C, FF), lambda i: (0, 0)),
            pl.BlockSpec((1, FF), lambda i: (0, 0)),
            pl.BlockSpec((FF, C), lambda i: (0, 0)),
            vec(), vec(), vec(), vec(), vec(),
        ],
        out_specs=pl.BlockSpec((BN, C), lambda i: (i, 0)),
        out_shape=jax.ShapeDtypeStruct((N, C), jnp.float32),
    )(accp, denp, x, bg, W1, b1, W2, b2, g1, be1, g2, be2)


def kernel(x, edge_index, Wg, att_src, att_dst, bg, W1, b1, W2, b2,
           g1, be1, g2, be2):
    loops = jnp.arange(N, dtype=jnp.int32)
    src = jnp.concatenate([edge_index[0], loops])
    dst = jnp.concatenate([edge_index[1], loops])
    src_p = jnp.concatenate(
        [src, jnp.zeros((EP - EE,), jnp.int32)]).reshape(-1, CHUNK)
    dst_p = jnp.concatenate(
        [dst, jnp.full((EP - EE,), N, jnp.int32)]).reshape(-1, CHUNK)

    h, adbuf, gbuf = _pre(x, Wg, att_src, att_dst)
    accp, denp = _edge(h.reshape(N * H, C), adbuf, gbuf, src_p, dst_p)
    return _post(accp, denp, x, bg.reshape(1, C), W1, b1.reshape(1, FF), W2,
                 b2.reshape(1, C), g1.reshape(1, C), be1.reshape(1, C),
                 g2.reshape(1, C), be2.reshape(1, C))


# speedup vs baseline: 1.0299x; 1.0299x over previous
import jax
import jax.numpy as jnp
from jax import lax
from jax.experimental import pallas as pl
from jax.experimental.pallas import tpu as pltpu
from jax.experimental.pallas import tpu_sc as plsc

N = 10000
C = 128
H = 4
FF = 512
E = 160000
EE = E + N
NTILES = 16
NCORES = 2
CHUNK = 64
CHUNKS_PER_TILE = 168
EP = NTILES * CHUNKS_PER_TILE * CHUNK
ACC_N = N + 112
ROWS_PER_TILE = ACC_N // NTILES
DW = 16
BN = 1000
GRID = N // BN


def _pre_body(x_ref, wg_ref, asrc_ref, adst_ref, h_ref, ad_ref, g_ref, macc_ref):
    i = pl.program_id(0)
    xb = x_ref[...]
    hb = jnp.dot(xb, wg_ref[...], preferred_element_type=jnp.float32)
    h_ref[...] = hb
    for hh in range(H):
        hcol = hb[:, hh * C:(hh + 1) * C]
        av = jnp.sum(hcol * asrc_ref[hh:hh + 1, :], axis=1)
        dv = jnp.sum(hcol * adst_ref[hh:hh + 1, :], axis=1)
        ad_ref[0, hh:hh + 1, :] = av.reshape(1, BN)
        ad_ref[0, H + hh:H + hh + 1, :] = dv.reshape(1, BN)
        ms = jnp.max(av)
        md = jnp.max(dv)

        @pl.when(i == 0)
        def _():
            macc_ref[hh:hh + 1, :] = jnp.full((1, 128), ms, jnp.float32)
            macc_ref[H + hh:H + hh + 1, :] = jnp.full((1, 128), md, jnp.float32)

        @pl.when(i > 0)
        def _():
            macc_ref[hh:hh + 1, :] = jnp.maximum(macc_ref[hh:hh + 1, :], ms)
            macc_ref[H + hh:H + hh + 1, :] = jnp.maximum(
                macc_ref[H + hh:H + hh + 1, :], md)

    @pl.when(i == GRID - 1)
    def _():
        g_ref[...] = jnp.concatenate(
            [jnp.maximum(macc_ref[0:H, :] + macc_ref[H:2 * H, :], 0.0),
             jnp.zeros((8 - H, 128), jnp.float32)], axis=0)


def _pre(x, Wg, att_src, att_dst):
    return pl.pallas_call(
        _pre_body,
        grid=(GRID,),
        in_specs=[
            pl.BlockSpec((BN, C), lambda i: (i, 0)),
            pl.BlockSpec((C, H * C), lambda i: (0, 0)),
            pl.BlockSpec((H, C), lambda i: (0, 0)),
            pl.BlockSpec((H, C), lambda i: (0, 0)),
        ],
        out_specs=[
            pl.BlockSpec((BN, H * C), lambda i: (i, 0)),
            pl.BlockSpec((1, 2 * H, BN), lambda i: (i, 0, 0)),
            pl.BlockSpec((2 * H, 128), lambda i: (0, 0)),
        ],
        out_shape=[
            jax.ShapeDtypeStruct((N, H * C), jnp.float32),
            jax.ShapeDtypeStruct((GRID, 2 * H, BN), jnp.float32),
            jax.ShapeDtypeStruct((2 * H, 128), jnp.float32),
        ],
        scratch_shapes=[pltpu.VMEM((2 * H, 128), jnp.float32)],
    )(x, Wg, att_src, att_dst)


def _edge_body(h40, adbuf, gbuf, srcI, dstI, accout, denout,
               as_l, ad_l, g_l, src_v, dst_v, idx_v, sidx_v, ea_v,
               rows_v, den_v, acc_sh, den_sh,
               gsem, isem_s, isem_d, ssem_a, ssem_d):
    c = lax.axis_index("c")
    s = lax.axis_index("s")
    onehot = jnp.where(lax.iota(jnp.int32, 16) == 0, 1.0, 0.0)
    lanes = [jnp.full((16,), i, jnp.int32) for i in range(16)]
    base = s * ROWS_PER_TILE
    CPT = CHUNKS_PER_TILE

    def _idx_compute(nb, hd):
        for k in range(CHUNK // 16):
            s16 = src_v[nb, pl.ds(16 * k, 16)]
            idx_v[nb, pl.ds(16 * k, 16)] = s16 * H + hd

    if True:
        for r in range(2):
            hd = r * NCORES + c
            for g in range(GRID):
                pltpu.sync_copy(adbuf.at[g].at[hd], as_l.at[pl.ds(g * BN, BN)])
                pltpu.sync_copy(adbuf.at[g].at[hd + H],
                                ad_l.at[pl.ds(g * BN, BN)])
            as_l[pl.ds(N, 16)] = jnp.zeros((16,), jnp.float32)
            ad_l[pl.ds(N, 16)] = jnp.zeros((16,), jnp.float32)
            pltpu.sync_copy(gbuf.at[hd], g_l)
            gvec = g_l[pl.ds(0, 16)]

            def zrow(rr, carry):
                for k in range(8):
                    rows_v[0, rr, pl.ds(16 * k, 16)] = jnp.zeros(
                        (16,), jnp.float32)
                den_v[0, rr, :] = jnp.zeros((16,), jnp.float32)
                return carry

            lax.fori_loop(0, CHUNK, zrow, 0)
            for k in range(ROWS_PER_TILE // CHUNK):
                pltpu.sync_copy(rows_v.at[0],
                                acc_sh.at[pl.ds(base + k * CHUNK, CHUNK)])
                pltpu.sync_copy(den_v.at[0],
                                den_sh.at[pl.ds(base + k * CHUNK, CHUNK)])
            rem = ROWS_PER_TILE % CHUNK
            pltpu.sync_copy(
                rows_v.at[0].at[pl.ds(0, rem)],
                acc_sh.at[pl.ds(base + ROWS_PER_TILE - rem, rem)])
            pltpu.sync_copy(
                den_v.at[0].at[pl.ds(0, rem)],
                den_sh.at[pl.ds(base + ROWS_PER_TILE - rem, rem)])
            plsc.subcore_barrier()

            pltpu.sync_copy(srcI.at[s * CPT], src_v.at[0])
            pltpu.sync_copy(dstI.at[s * CPT], dst_v.at[0])
            _idx_compute(0, hd)
            pltpu.async_copy(h40.at[idx_v.at[0]], rows_v.at[0], gsem.at[0])
            pltpu.async_copy(srcI.at[s * CPT + 1], src_v.at[1], isem_s.at[1])
            pltpu.async_copy(dstI.at[s * CPT + 1], dst_v.at[1], isem_d.at[1])

            def chunk_body(j, carry):
                bb = lax.rem(j, 2)
                nb = 1 - bb
                pltpu.make_async_copy(
                    h40.at[idx_v.at[bb]], rows_v.at[bb], gsem.at[bb]).wait()

                @pl.when(j < CPT - 1)
                def _():
                    pltpu.make_async_copy(
                        srcI.at[0], src_v.at[nb], isem_s.at[nb]).wait()
                    pltpu.make_async_copy(
                        dstI.at[0], dst_v.at[nb], isem_d.at[nb]).wait()
                    _idx_compute(nb, hd)

                    @pl.when(j > 0)
                    def _():
                        pltpu.make_async_copy(
                            rows_v.at[nb], acc_sh.at[sidx_v.at[nb]],
                            ssem_a.at[nb]).wait()
                        pltpu.make_async_copy(
                            den_v.at[nb], den_sh.at[sidx_v.at[nb]],
                            ssem_d.at[nb]).wait()

                    pltpu.async_copy(h40.at[idx_v.at[nb]], rows_v.at[nb],
                                     gsem.at[nb])

                for g2 in range(CHUNK // 16):
                    s16 = src_v[bb, pl.ds(16 * g2, 16)]
                    d16 = dst_v[bb, pl.ds(16 * g2, 16)]
                    al = (plsc.load_gather(as_l, [s16])
                          + plsc.load_gather(ad_l, [d16]))
                    al = jnp.maximum(al, al * 0.2)
                    eavec = jnp.exp(al - gvec)
                    sidx_v[bb, pl.ds(16 * g2, 16)] = d16
                    for i in range(16):
                        rr = g2 * 16 + i
                        eav = jnp.full((16,), eavec[i], jnp.float32)
                        for k in range(8):
                            rows_v[bb, rr, pl.ds(16 * k, 16)] = (
                                rows_v[bb, rr, pl.ds(16 * k, 16)] * eav)
                        den_v[bb, rr, :] = eav * onehot
                pltpu.async_copy(rows_v.at[bb], acc_sh.at[sidx_v.at[bb]],
                                 ssem_a.at[bb], add=True)
                pltpu.async_copy(den_v.at[bb], den_sh.at[sidx_v.at[bb]],
                                 ssem_d.at[bb], add=True)
                @pl.when(j < CPT - 2)
                def _():
                    row2 = s * CPT + j + 2
                    pltpu.async_copy(srcI.at[row2], src_v.at[bb],
                                     isem_s.at[bb])
                    pltpu.async_copy(dstI.at[row2], dst_v.at[bb],
                                     isem_d.at[bb])
                return carry

            lax.fori_loop(0, CPT, chunk_body, 0)
            for p in ((CPT - 2) % 2, (CPT - 1) % 2):
                pltpu.make_async_copy(
                    rows_v.at[p], acc_sh.at[sidx_v.at[p]], ssem_a.at[p]).wait()
                pltpu.make_async_copy(
                    den_v.at[p], den_sh.at[sidx_v.at[p]], ssem_d.at[p]).wait()
            plsc.subcore_barrier()
            pltpu.sync_copy(acc_sh.at[pl.ds(base, ROWS_PER_TILE)],
                            accout.at[hd].at[pl.ds(base, ROWS_PER_TILE)])
            pltpu.sync_copy(den_sh.at[pl.ds(base, ROWS_PER_TILE)],
                            denout.at[hd].at[pl.ds(base, ROWS_PER_TILE)])
            plsc.subcore_barrier()


def _edge(h40, adbuf, gbuf, src_p, dst_p):
    mesh = plsc.VectorSubcoreMesh(core_axis_name="c", subcore_axis_name="s",
                                  num_cores=NCORES, num_subcores=NTILES)
    fn = pl.kernel(
        _edge_body,
        out_type=[
            jax.ShapeDtypeStruct((H, ACC_N, C), jnp.float32),
            jax.ShapeDtypeStruct((H, ACC_N, DW), jnp.float32),
        ],
        mesh=mesh,
        compiler_params=pltpu.CompilerParams(
            needs_layout_passes=False, use_tc_tiling_on_sc=False),
        scratch_types=[
            pltpu.VMEM((N + 16,), jnp.float32),
            pltpu.VMEM((N + 16,), jnp.float32),
            pltpu.VMEM((128,), jnp.float32),
            pltpu.VMEM((2, CHUNK), jnp.int32),
            pltpu.VMEM((2, CHUNK), jnp.int32),
            pltpu.VMEM((2, CHUNK), jnp.int32),
            pltpu.VMEM((2, CHUNK), jnp.int32),
            pltpu.VMEM((CHUNK,), jnp.float32),
            pltpu.VMEM((2, CHUNK, C), jnp.float32),
            pltpu.VMEM((2, CHUNK, DW), jnp.float32),
            pltpu.VMEM_SHARED((ACC_N, C), jnp.float32),
            pltpu.VMEM_SHARED((ACC_N, DW), jnp.float32),
            pltpu.SemaphoreType.DMA((2,)),
            pltpu.SemaphoreType.DMA((2,)),
            pltpu.SemaphoreType.DMA((2,)),
            pltpu.SemaphoreType.DMA((2,)),
            pltpu.SemaphoreType.DMA((2,)),
        ],
    )
    return fn(h40, adbuf, gbuf, src_p, dst_p)


def _post_body(acc_ref, den_ref, x_ref, bg_ref, w1_ref, b1_ref, w2_ref,
               b2_ref, g1_ref, be1_ref, g2_ref, be2_ref, o_ref):
    num = acc_ref[...]
    den = den_ref[:, :, 0:1]
    outv = jnp.sum(num / (den + 1e-16), axis=0) * (1.0 / H) + bg_ref[...]
    t = x_ref[...] + outv
    mu = jnp.mean(t, axis=-1, keepdims=True)
    var = jnp.mean((t - mu) ** 2, axis=-1, keepdims=True)
    x1 = (t - mu) / jnp.sqrt(var + 1e-5) * g1_ref[...] + be1_ref[...]
    f = jnp.maximum(
        jnp.dot(x1, w1_ref[...], preferred_element_type=jnp.float32)
        + b1_ref[...], 0.0)
    f2 = (jnp.dot(f, w2_ref[...], preferred_element_type=jnp.float32)
          + b2_ref[...])
    t2 = x1 + f2
    mu2 = jnp.mean(t2, axis=-1, keepdims=True)
    var2 = jnp.mean((t2 - mu2) ** 2, axis=-1, keepdims=True)
    o_ref[...] = (t2 - mu2) / jnp.sqrt(var2 + 1e-5) * g2_ref[...] + be2_ref[...]


def _post(accp, denp, x, bg, W1, b1, W2, b2, g1, be1, g2, be2):
    def vec():
        return pl.BlockSpec((1, C), lambda i: (0, 0))
    return pl.pallas_call(
        _post_body,
        grid=(GRID,),
        in_specs=[
            pl.BlockSpec((H, BN, C), lambda i: (0, i, 0)),
            pl.BlockSpec((H, BN, DW), lambda i: (0, i, 0)),
            pl.BlockSpec((BN, C), lambda i: (i, 0)),
            vec(),
            pl.BlockSpec((C, FF), lambda i: (0, 0)),
            pl.BlockSpec((1, FF), lambda i: (0, 0)),
            pl.BlockSpec((FF, C), lambda i: (0, 0)),
            vec(), vec(), vec(), vec(), vec(),
        ],
        out_specs=pl.BlockSpec((BN, C), lambda i: (i, 0)),
        out_shape=jax.ShapeDtypeStruct((N, C), jnp.float32),
    )(accp, denp, x, bg, W1, b1, W2, b2, g1, be1, g2, be2)


def kernel(x, edge_index, Wg, att_src, att_dst, bg, W1, b1, W2, b2,
           g1, be1, g2, be2):
    loops = jnp.arange(N, dtype=jnp.int32)
    src = jnp.concatenate([edge_index[0], loops])
    dst = jnp.concatenate([edge_index[1], loops])
    src_p = jnp.concatenate(
        [src, jnp.zeros((EP - EE,), jnp.int32)]).reshape(-1, CHUNK)
    dst_p = jnp.concatenate(
        [dst, jnp.full((EP - EE,), N, jnp.int32)]).reshape(-1, CHUNK)

    h, adbuf, gbuf = _pre(x, Wg, att_src, att_dst)
    accp, denp = _edge(h.reshape(N * H, C), adbuf, gbuf, src_p, dst_p)
    return _post(accp, denp, x, bg.reshape(1, C), W1, b1.reshape(1, FF), W2,
                 b2.reshape(1, C), g1.reshape(1, C), be1.reshape(1, C),
                 g2.reshape(1, C), be2.reshape(1, C))
